# baseline (speedup 1.0000x reference)
import functools
import math

import jax
import jax.numpy as jnp
from jax.experimental import pallas as pl
from jax.experimental.pallas import tpu as pltpu

_INV_SQRT2 = 1.0 / math.sqrt(2.0)


def _router_block(x_ref, w1_ref, b1_ref, w2_ref, b2_ref, packets_ref, probs_ref):
    h = jnp.dot(x_ref[...], w1_ref[...], preferred_element_type=jnp.float32)
    h = h + b1_ref[...]
    h = 0.5 * h * (1.0 + jax.lax.erf(h * _INV_SQRT2))
    logits = jnp.dot(h, w2_ref[...], preferred_element_type=jnp.float32)
    logits = logits + b2_ref[...]
    packets_ref[...] = jnp.argmax(logits, axis=-1, keepdims=True).astype(jnp.int32)
    m = jnp.max(logits, axis=-1, keepdims=True)
    e = jnp.exp(logits - m)
    probs_ref[...] = e / jnp.sum(e, axis=-1, keepdims=True)


@functools.partial(jax.jit, static_argnames=("block_n",))
def kernel(x, W1, b1, W2, b2, block_n: int = 2048):
    n, d = x.shape
    h_dim = W1.shape[1]
    p = W2.shape[1]
    grid = (n // block_n,)
    packets2d, probs = pl.pallas_call(
        _router_block,
        grid=grid,
        in_specs=[
            pl.BlockSpec((block_n, d), lambda i: (i, 0)),
            pl.BlockSpec((d, h_dim), lambda i: (0, 0)),
            pl.BlockSpec((h_dim,), lambda i: (0,)),
            pl.BlockSpec((h_dim, p), lambda i: (0, 0)),
            pl.BlockSpec((p,), lambda i: (0,)),
        ],
        out_specs=[
            pl.BlockSpec((block_n, 1), lambda i: (i, 0)),
            pl.BlockSpec((block_n, p), lambda i: (i, 0)),
        ],
        out_shape=[
            jax.ShapeDtypeStruct((n, 1), jnp.int32),
            jax.ShapeDtypeStruct((n, p), jnp.float32),
        ],
        compiler_params=pltpu.CompilerParams(
            dimension_semantics=("arbitrary",),
        ),
    )(x, W1, b1, W2, b2)
    return packets2d.reshape(n), probs
